# baseline (speedup 1.0000x reference)
import functools

import jax
import jax.numpy as jnp
from jax import lax
from jax.experimental import pallas as pl
from jax.experimental.pallas import tpu as pltpu
from jax.experimental.pallas import tpu_sc as plsc

_CHUNK_ROWS = 16


def kernel(x, weight):
    seq_len = x.shape[-2]
    dim = weight.shape[1]
    info = plsc.get_sparse_core_info()
    num_tiles = info.num_cores * info.num_subcores
    rows_per_tile = seq_len // num_tiles
    nchunks = rows_per_tile // _CHUNK_ROWS
    mesh = plsc.VectorSubcoreMesh(core_axis_name="c", subcore_axis_name="s")

    @functools.partial(
        pl.kernel,
        out_type=jax.ShapeDtypeStruct((seq_len, dim), weight.dtype),
        mesh=mesh,
        scratch_types=[pltpu.VMEM((_CHUNK_ROWS, dim), jnp.float32)],
    )
    def _stripe_copy(w_hbm, o_hbm, buf):
        tile = lax.axis_index("s") * info.num_cores + lax.axis_index("c")
        base = tile * rows_per_tile

        def body(g, carry):
            start = base + g * _CHUNK_ROWS
            pltpu.sync_copy(w_hbm.at[pl.ds(start, _CHUNK_ROWS)], buf)
            pltpu.sync_copy(buf, o_hbm.at[pl.ds(start, _CHUNK_ROWS)])
            return carry

        lax.fori_loop(0, nchunks, body, 0)

    return _stripe_copy(weight)


# speedup vs baseline: 1.1472x; 1.1472x over previous
import functools

import jax
import jax.numpy as jnp
from jax import lax
from jax.experimental import pallas as pl
from jax.experimental.pallas import tpu as pltpu
from jax.experimental.pallas import tpu_sc as plsc

_CHUNK_ROWS = 16


def kernel(x, weight):
    seq_len = x.shape[-2]
    dim = weight.shape[1]
    info = plsc.get_sparse_core_info()
    num_tiles = info.num_cores * info.num_subcores
    rows_per_tile = seq_len // num_tiles
    nchunks = rows_per_tile // _CHUNK_ROWS
    mesh = plsc.VectorSubcoreMesh(core_axis_name="c", subcore_axis_name="s")

    @functools.partial(
        pl.kernel,
        out_type=jax.ShapeDtypeStruct((seq_len, dim), weight.dtype),
        mesh=mesh,
        scratch_types=[
            pltpu.VMEM((2, _CHUNK_ROWS, dim), jnp.float32),
            pltpu.SemaphoreType.DMA((2,)),
            pltpu.SemaphoreType.DMA((2,)),
        ],
    )
    def _stripe_copy(w_hbm, o_hbm, buf, in_sems, out_sems):
        tile = lax.axis_index("s") * info.num_cores + lax.axis_index("c")
        base = tile * rows_per_tile

        def in_copy(j):
            return pltpu.make_async_copy(
                w_hbm.at[pl.ds(base + j * _CHUNK_ROWS, _CHUNK_ROWS)],
                buf.at[j % 2],
                in_sems.at[j % 2],
            )

        def out_copy(j):
            return pltpu.make_async_copy(
                buf.at[j % 2],
                o_hbm.at[pl.ds(base + j * _CHUNK_ROWS, _CHUNK_ROWS)],
                out_sems.at[j % 2],
            )

        in_copy(0).start()
        for j in range(nchunks):
            if j + 1 < nchunks:
                if j >= 1:
                    out_copy(j - 1).wait()
                in_copy(j + 1).start()
            in_copy(j).wait()
            out_copy(j).start()
        out_copy(nchunks - 1).wait()
        if nchunks >= 2:
            out_copy(nchunks - 2).wait()

    return _stripe_copy(weight)


# speedup vs baseline: 1.1575x; 1.0090x over previous
import functools

import jax
import jax.numpy as jnp
from jax import lax
from jax.experimental import pallas as pl
from jax.experimental.pallas import tpu as pltpu
from jax.experimental.pallas import tpu_sc as plsc

_CHUNK_ROWS = 16
_NBUF = 3


def kernel(x, weight):
    seq_len = x.shape[-2]
    dim = weight.shape[1]
    info = plsc.get_sparse_core_info()
    num_tiles = info.num_cores * info.num_subcores
    rows_per_tile = seq_len // num_tiles
    nchunks = rows_per_tile // _CHUNK_ROWS
    mesh = plsc.VectorSubcoreMesh(core_axis_name="c", subcore_axis_name="s")

    @functools.partial(
        pl.kernel,
        out_type=jax.ShapeDtypeStruct((seq_len, dim), weight.dtype),
        mesh=mesh,
        scratch_types=[
            pltpu.VMEM((_NBUF, _CHUNK_ROWS, dim), jnp.float32),
            pltpu.SemaphoreType.DMA((_NBUF,)),
            pltpu.SemaphoreType.DMA((_NBUF,)),
        ],
    )
    def _stripe_copy(w_hbm, o_hbm, buf, in_sems, out_sems):
        tile = lax.axis_index("s") * info.num_cores + lax.axis_index("c")
        base = tile * rows_per_tile

        def in_copy(j):
            return pltpu.make_async_copy(
                w_hbm.at[pl.ds(base + j * _CHUNK_ROWS, _CHUNK_ROWS)],
                buf.at[j % _NBUF],
                in_sems.at[j % _NBUF],
            )

        def out_copy(j):
            return pltpu.make_async_copy(
                buf.at[j % _NBUF],
                o_hbm.at[pl.ds(base + j * _CHUNK_ROWS, _CHUNK_ROWS)],
                out_sems.at[j % _NBUF],
            )

        for j in range(min(_NBUF - 1, nchunks)):
            in_copy(j).start()
        for j in range(nchunks):
            nxt = j + _NBUF - 1
            if nxt < nchunks:
                if nxt - _NBUF >= 0:
                    out_copy(nxt - _NBUF).wait()
                in_copy(nxt).start()
            in_copy(j).wait()
            out_copy(j).start()
        for j in range(max(nchunks - _NBUF, 0), nchunks):
            out_copy(j).wait()

    return _stripe_copy(weight)


# speedup vs baseline: 1.7440x; 1.5067x over previous
import jax
import jax.numpy as jnp
from jax.experimental import pallas as pl
from jax.experimental.pallas import tpu as pltpu

_BLK = 1024
_NBUF = 2


def _copy_body(w_hbm, o_hbm):
    nblk = o_hbm.shape[0] // _BLK
    dim = o_hbm.shape[1]

    def scoped(buf, in_sems, out_sems):
        def in_copy(j):
            return pltpu.make_async_copy(
                w_hbm.at[pl.ds(j * _BLK, _BLK)],
                buf.at[j % _NBUF],
                in_sems.at[j % _NBUF],
            )

        def out_copy(j):
            return pltpu.make_async_copy(
                buf.at[j % _NBUF],
                o_hbm.at[pl.ds(j * _BLK, _BLK)],
                out_sems.at[j % _NBUF],
            )

        for j in range(min(_NBUF - 1, nblk)):
            in_copy(j).start()
        for j in range(nblk):
            nxt = j + _NBUF - 1
            if nxt < nblk:
                if nxt - _NBUF >= 0:
                    out_copy(nxt - _NBUF).wait()
                in_copy(nxt).start()
            in_copy(j).wait()
            out_copy(j).start()
        for j in range(max(nblk - _NBUF, 0), nblk):
            out_copy(j).wait()

    pl.run_scoped(
        scoped,
        pltpu.VMEM((_NBUF, _BLK, dim), jnp.float32),
        pltpu.SemaphoreType.DMA((_NBUF,)),
        pltpu.SemaphoreType.DMA((_NBUF,)),
    )


def kernel(x, weight):
    seq_len = x.shape[-2]
    dim = weight.shape[1]
    return pl.pallas_call(
        _copy_body,
        out_shape=jax.ShapeDtypeStruct((seq_len, dim), weight.dtype),
        in_specs=[pl.BlockSpec(memory_space=pl.ANY)],
        out_specs=pl.BlockSpec(memory_space=pl.ANY),
    )(weight)


# speedup vs baseline: 1.7692x; 1.0145x over previous
import jax
import jax.numpy as jnp
from jax.experimental import pallas as pl
from jax.experimental.pallas import tpu as pltpu

_BLK = 512
_NBUF = 4


def _copy_body(w_hbm, o_hbm):
    nblk = o_hbm.shape[0] // _BLK
    dim = o_hbm.shape[1]

    def scoped(buf, in_sems, out_sems):
        def in_copy(j):
            return pltpu.make_async_copy(
                w_hbm.at[pl.ds(j * _BLK, _BLK)],
                buf.at[j % _NBUF],
                in_sems.at[j % _NBUF],
            )

        def out_copy(j):
            return pltpu.make_async_copy(
                buf.at[j % _NBUF],
                o_hbm.at[pl.ds(j * _BLK, _BLK)],
                out_sems.at[j % _NBUF],
            )

        for j in range(min(_NBUF - 1, nblk)):
            in_copy(j).start()
        for j in range(nblk):
            nxt = j + _NBUF - 1
            if nxt < nblk:
                if nxt - _NBUF >= 0:
                    out_copy(nxt - _NBUF).wait()
                in_copy(nxt).start()
            in_copy(j).wait()
            out_copy(j).start()
        for j in range(max(nblk - _NBUF, 0), nblk):
            out_copy(j).wait()

    pl.run_scoped(
        scoped,
        pltpu.VMEM((_NBUF, _BLK, dim), jnp.float32),
        pltpu.SemaphoreType.DMA((_NBUF,)),
        pltpu.SemaphoreType.DMA((_NBUF,)),
    )


def kernel(x, weight):
    seq_len = x.shape[-2]
    dim = weight.shape[1]
    return pl.pallas_call(
        _copy_body,
        out_shape=jax.ShapeDtypeStruct((seq_len, dim), weight.dtype),
        in_specs=[pl.BlockSpec(memory_space=pl.ANY)],
        out_specs=pl.BlockSpec(memory_space=pl.ANY),
    )(weight)


# speedup vs baseline: 1.7807x; 1.0065x over previous
import jax
import jax.numpy as jnp
from jax.experimental import pallas as pl
from jax.experimental.pallas import tpu as pltpu

_BLOCK_ROWS = 1024


def _slice_copy_kernel(w_ref, o_ref):
    o_ref[...] = w_ref[...]


def kernel(x, weight):
    seq_len = x.shape[-2]
    dim = weight.shape[1]
    grid = (seq_len // _BLOCK_ROWS,)
    return pl.pallas_call(
        _slice_copy_kernel,
        out_shape=jax.ShapeDtypeStruct((seq_len, dim), weight.dtype),
        grid=grid,
        in_specs=[pl.BlockSpec((_BLOCK_ROWS, dim), lambda i: (i, 0))],
        out_specs=pl.BlockSpec((_BLOCK_ROWS, dim), lambda i: (i, 0)),
    )(weight)
